# Initial kernel scaffold; baseline (speedup 1.0000x reference)
import jax
import jax.numpy as jnp
from jax.experimental import pallas as pl


def kernel(x, edge_index, W1l, b1l, W1r, W2l, b2l, W2r, Wc, bc):
    raise NotImplementedError("write your pallas kernel here")


# speedup vs baseline: 43.0133x; 43.0133x over previous
import functools

import jax
import jax.numpy as jnp
from jax import lax
from jax.experimental import pallas as pl
from jax.experimental.pallas import tpu as pltpu
from jax.experimental.pallas import tpu_sc as plsc

_N = 10000
_E = 320000
_NR = 80
_NSUB = 16
_EPT = _E // _NSUB
_NITER = _EPT // 16
_SPT = _NR // _NSUB


def _sc_segsums(src, dst, zeros):
    mesh = plsc.VectorSubcoreMesh(core_axis_name="c", subcore_axis_name="s")

    @functools.partial(
        pl.kernel,
        out_type=[
            jax.ShapeDtypeStruct((_NR, 128), jnp.float32),
            jax.ShapeDtypeStruct((_NR, 128), jnp.float32),
        ],
        mesh=mesh,
        compiler_params=pltpu.CompilerParams(needs_layout_passes=False),
        scratch_types=[
            pltpu.VMEM((_EPT,), jnp.int32),
            pltpu.VMEM((_EPT,), jnp.int32),
            pltpu.VMEM((_NR, 128), jnp.float32),
            pltpu.VMEM((_NR, 128), jnp.float32),
            pltpu.VMEM((_SPT, 128), jnp.float32),
            pltpu.VMEM((_SPT, 128), jnp.float32),
            pltpu.VMEM((_NR,), jnp.int32),
            pltpu.VMEM_SHARED((_NR, 128), jnp.float32),
        ],
    )
    def body(src_hbm, dst_hbm, zeros_hbm, u_out, w_out,
             src_t, dst_t, tbl, acc, t5a, t5b, rows, sh):
        c = lax.axis_index("c")
        s = lax.axis_index("s")
        stripe = pl.ds(s * _SPT, _SPT)
        is_leader = jnp.logical_and(s == 0, c == 0)

        for b in range(_NR // 16):
            rows[pl.ds(b * 16, 16)] = lax.iota(jnp.int32, 16) + 16 * b

        pltpu.sync_copy(src_hbm.at[pl.ds(s * _EPT, _EPT)], src_t)
        pltpu.sync_copy(dst_hbm.at[pl.ds(s * _EPT, _EPT)], dst_t)
        pltpu.sync_copy(zeros_hbm, acc)

        @pl.when(s == 0)
        def _():
            pltpu.sync_copy(zeros_hbm, sh)

        plsc.subcore_barrier()

        ones16 = jnp.ones((16,), jnp.float32)

        def p_cnt(k, carry):
            dv = dst_t[pl.ds(k * 16, 16)]
            plsc.addupdate_scatter(acc, [dv >> 7, dv & 127], ones16)
            return carry

        lax.fori_loop(0, _NITER, p_cnt, 0)
        pltpu.sync_copy(acc, sh.at[rows], add=True)
        plsc.subcore_barrier()

        pltpu.sync_copy(sh.at[stripe], t5a)
        for i in range(_SPT):
            for j in range(8):
                v = t5a[i, pl.ds(j * 16, 16)]
                rv = 1.0 / jnp.maximum(v, 1.0)
                t5a[i, pl.ds(j * 16, 16)] = rv
                t5b[i, pl.ds(j * 16, 16)] = rv
        pltpu.sync_copy(t5a, sh.at[stripe])
        plsc.subcore_barrier()
        pltpu.sync_copy(sh, tbl)
        pltpu.sync_copy(zeros_hbm, acc)
        plsc.subcore_barrier()

        @pl.when(s == 0)
        def _():
            pltpu.sync_copy(zeros_hbm, sh)

        plsc.subcore_barrier()

        def p_gather(k, carry):
            dv = dst_t[pl.ds(k * 16, 16)]
            sv = src_t[pl.ds(k * 16, 16)]
            vals = plsc.load_gather(tbl, [dv >> 7, dv & 127])
            plsc.addupdate_scatter(acc, [sv >> 7, sv & 127], vals)
            return carry

        lax.fori_loop(0, _NITER, p_gather, 0)
        pltpu.sync_copy(acc, sh.at[rows], add=True)
        plsc.subcore_barrier()

        @pl.when(is_leader)
        def _():
            pltpu.sync_copy(sh, u_out)

        pltpu.sync_copy(sh.at[stripe], t5a)
        for i in range(_SPT):
            for j in range(8):
                t5a[i, pl.ds(j * 16, 16)] = (
                    t5a[i, pl.ds(j * 16, 16)] * t5b[i, pl.ds(j * 16, 16)]
                )
        plsc.subcore_barrier()
        pltpu.sync_copy(t5a, sh.at[stripe])
        plsc.subcore_barrier()
        pltpu.sync_copy(sh, tbl)
        pltpu.sync_copy(zeros_hbm, acc)
        plsc.subcore_barrier()

        @pl.when(s == 0)
        def _():
            pltpu.sync_copy(zeros_hbm, sh)

        plsc.subcore_barrier()

        lax.fori_loop(0, _NITER, p_gather, 0)
        pltpu.sync_copy(acc, sh.at[rows], add=True)
        plsc.subcore_barrier()

        @pl.when(is_leader)
        def _():
            pltpu.sync_copy(sh, w_out)

    return body(src, dst, zeros)


def _tc_final(uw, x, W1l, b1l, W1r, W2l, b2l, W2r, Wc, bc):
    def body(uw_ref, x_ref, w1l_ref, b1l_ref, w1r_ref, w2l_ref, b2l_ref,
             w2r_ref, wc_ref, bc_ref, out_ref):
        f32 = jnp.float32
        R = jnp.dot(uw_ref[...], x_ref[...], preferred_element_type=f32)
        A = jnp.dot(R, w1l_ref[...], preferred_element_type=f32)
        B = jnp.dot(R, w1r_ref[...], preferred_element_type=f32)
        sum_u = jnp.sum(uw_ref[1:2, :])
        b1 = b1l_ref[...]
        sum_h1 = A[1:2, :] + _N * b1 + B[0:1, :]
        uth1 = A[2:3, :] + sum_u * b1 + B[1:2, :]
        T = jnp.concatenate([sum_h1, uth1], axis=0)
        C2 = jnp.dot(T, w2l_ref[...], preferred_element_type=f32)
        D2 = jnp.dot(T, w2r_ref[...], preferred_element_type=f32)
        sum_h2 = C2[1:2, :] + _N * b2l_ref[...] + D2[0:1, :]
        out_ref[...] = (
            jnp.dot(sum_h2 * (1.0 / _N), wc_ref[...], preferred_element_type=f32)
            + bc_ref[...]
        )

    return pl.pallas_call(
        body,
        out_shape=jax.ShapeDtypeStruct((1, 64), jnp.float32),
    )(uw, x, W1l, b1l, W1r, W2l, b2l, W2r, Wc, bc)


def kernel(x, edge_index, W1l, b1l, W1r, W2l, b2l, W2r, Wc, bc):
    src = edge_index[0]
    dst = edge_index[1]
    zeros = jnp.zeros((_NR, 128), jnp.float32)
    u2d, w2d = _sc_segsums(src, dst, zeros)
    u = u2d.reshape(_NR * 128)[:_N]
    w = w2d.reshape(_NR * 128)[:_N]
    uw = jnp.concatenate(
        [
            jnp.ones((1, _N), jnp.float32),
            u[None, :],
            w[None, :],
            jnp.zeros((5, _N), jnp.float32),
        ],
        axis=0,
    )
    return _tc_final(uw, x, W1l, b1l[None, :], W1r, W2l, b2l[None, :], W2r,
                     Wc, bc[None, :])


# speedup vs baseline: 58.9244x; 1.3699x over previous
import functools

import jax
import jax.numpy as jnp
from jax import lax
from jax.experimental import pallas as pl
from jax.experimental.pallas import tpu as pltpu
from jax.experimental.pallas import tpu_sc as plsc

_N = 10000
_E = 320000
_NR = 80
_NSUB = 16
_EPT = _E // _NSUB
_NITER = _EPT // 16
_SPT = _NR // _NSUB


def _sc_segsums(src, dst, zeros):
    mesh = plsc.VectorSubcoreMesh(
        core_axis_name="c", subcore_axis_name="s", num_cores=1
    )

    @functools.partial(
        pl.kernel,
        out_type=[
            jax.ShapeDtypeStruct((_NR, 128), jnp.float32),
            jax.ShapeDtypeStruct((_NR, 128), jnp.float32),
        ],
        mesh=mesh,
        compiler_params=pltpu.CompilerParams(needs_layout_passes=False),
        scratch_types=[
            pltpu.VMEM((_EPT,), jnp.int32),
            pltpu.VMEM((_EPT,), jnp.int32),
            pltpu.VMEM((_NR, 128), jnp.float32),
            pltpu.VMEM((_NR, 128), jnp.float32),
            pltpu.VMEM((_SPT, 128), jnp.float32),
            pltpu.VMEM((_SPT, 128), jnp.float32),
            pltpu.VMEM((_NR,), jnp.int32),
            pltpu.VMEM_SHARED((_NR, 128), jnp.float32),
        ],
    )
    def body(src_hbm, dst_hbm, zeros_hbm, u_out, w_out,
             src_t, dst_t, tbl, acc, t5a, t5b, rows, sh):
        c = lax.axis_index("c")
        s = lax.axis_index("s")
        stripe = pl.ds(s * _SPT, _SPT)
        is_leader = jnp.logical_and(s == 0, c == 0)

        for b in range(_NR // 16):
            rows[pl.ds(b * 16, 16)] = lax.iota(jnp.int32, 16) + 16 * b

        pltpu.sync_copy(src_hbm.at[pl.ds(s * _EPT, _EPT)], src_t)
        pltpu.sync_copy(dst_hbm.at[pl.ds(s * _EPT, _EPT)], dst_t)
        pltpu.sync_copy(zeros_hbm, acc)

        @pl.when(s == 0)
        def _():
            pltpu.sync_copy(zeros_hbm, sh)

        plsc.subcore_barrier()

        ones16 = jnp.ones((16,), jnp.float32)

        @plsc.parallel_loop(0, _NITER, 1, unroll=5)
        def _(k):
            dv = dst_t[pl.ds(k * 16, 16)]
            plsc.addupdate_scatter(acc, [dv >> 7, dv & 127], ones16)
        pltpu.sync_copy(acc, sh.at[rows], add=True)
        plsc.subcore_barrier()

        pltpu.sync_copy(sh.at[stripe], t5a)
        for i in range(_SPT):
            for j in range(8):
                v = t5a[i, pl.ds(j * 16, 16)]
                rv = 1.0 / jnp.maximum(v, 1.0)
                t5a[i, pl.ds(j * 16, 16)] = rv
                t5b[i, pl.ds(j * 16, 16)] = rv
        pltpu.sync_copy(t5a, sh.at[stripe])
        plsc.subcore_barrier()
        pltpu.sync_copy(sh, tbl)
        pltpu.sync_copy(zeros_hbm, acc)
        plsc.subcore_barrier()

        @pl.when(s == 0)
        def _():
            pltpu.sync_copy(zeros_hbm, sh)

        plsc.subcore_barrier()

        def p_gather(k):
            dv = dst_t[pl.ds(k * 16, 16)]
            sv = src_t[pl.ds(k * 16, 16)]
            vals = plsc.load_gather(tbl, [dv >> 7, dv & 127])
            plsc.addupdate_scatter(acc, [sv >> 7, sv & 127], vals)

        plsc.parallel_loop(0, _NITER, 1, unroll=5)(p_gather)
        pltpu.sync_copy(acc, sh.at[rows], add=True)
        plsc.subcore_barrier()

        @pl.when(is_leader)
        def _():
            pltpu.sync_copy(sh, u_out)

        pltpu.sync_copy(sh.at[stripe], t5a)
        for i in range(_SPT):
            for j in range(8):
                t5a[i, pl.ds(j * 16, 16)] = (
                    t5a[i, pl.ds(j * 16, 16)] * t5b[i, pl.ds(j * 16, 16)]
                )
        plsc.subcore_barrier()
        pltpu.sync_copy(t5a, sh.at[stripe])
        plsc.subcore_barrier()
        pltpu.sync_copy(sh, tbl)
        pltpu.sync_copy(zeros_hbm, acc)
        plsc.subcore_barrier()

        @pl.when(s == 0)
        def _():
            pltpu.sync_copy(zeros_hbm, sh)

        plsc.subcore_barrier()

        plsc.parallel_loop(0, _NITER, 1, unroll=5)(p_gather)
        pltpu.sync_copy(acc, sh.at[rows], add=True)
        plsc.subcore_barrier()

        @pl.when(is_leader)
        def _():
            pltpu.sync_copy(sh, w_out)

    return body(src, dst, zeros)


def _tc_final(uw, x, W1l, b1l, W1r, W2l, b2l, W2r, Wc, bc):
    def body(uw_ref, x_ref, w1l_ref, b1l_ref, w1r_ref, w2l_ref, b2l_ref,
             w2r_ref, wc_ref, bc_ref, out_ref):
        f32 = jnp.float32
        R = jnp.dot(uw_ref[...], x_ref[...], preferred_element_type=f32, precision=lax.Precision.HIGHEST)
        A = jnp.dot(R, w1l_ref[...], preferred_element_type=f32, precision=lax.Precision.HIGHEST)
        B = jnp.dot(R, w1r_ref[...], preferred_element_type=f32, precision=lax.Precision.HIGHEST)
        sum_u = jnp.sum(uw_ref[1:2, :])
        b1 = b1l_ref[...]
        sum_h1 = A[1:2, :] + _N * b1 + B[0:1, :]
        uth1 = A[2:3, :] + sum_u * b1 + B[1:2, :]
        T = jnp.concatenate([sum_h1, uth1], axis=0)
        C2 = jnp.dot(T, w2l_ref[...], preferred_element_type=f32, precision=lax.Precision.HIGHEST)
        D2 = jnp.dot(T, w2r_ref[...], preferred_element_type=f32, precision=lax.Precision.HIGHEST)
        sum_h2 = C2[1:2, :] + _N * b2l_ref[...] + D2[0:1, :]
        out_ref[...] = (
            jnp.dot(sum_h2 * (1.0 / _N), wc_ref[...], preferred_element_type=f32, precision=lax.Precision.HIGHEST)
            + bc_ref[...]
        )

    return pl.pallas_call(
        body,
        out_shape=jax.ShapeDtypeStruct((1, 64), jnp.float32),
    )(uw, x, W1l, b1l, W1r, W2l, b2l, W2r, Wc, bc)


def kernel(x, edge_index, W1l, b1l, W1r, W2l, b2l, W2r, Wc, bc):
    src = edge_index[0]
    dst = edge_index[1]
    zeros = jnp.zeros((_NR, 128), jnp.float32)
    u2d, w2d = _sc_segsums(src, dst, zeros)
    u = u2d.reshape(_NR * 128)[:_N]
    w = w2d.reshape(_NR * 128)[:_N]
    uw = jnp.concatenate(
        [
            jnp.ones((1, _N), jnp.float32),
            u[None, :],
            w[None, :],
            jnp.zeros((5, _N), jnp.float32),
        ],
        axis=0,
    )
    return _tc_final(uw, x, W1l, b1l[None, :], W1r, W2l, b2l[None, :], W2r,
                     Wc, bc[None, :])


# speedup vs baseline: 72.9409x; 1.2379x over previous
import functools

import jax
import jax.numpy as jnp
from jax import lax
from jax.experimental import pallas as pl
from jax.experimental.pallas import tpu as pltpu
from jax.experimental.pallas import tpu_sc as plsc

_N = 10000
_E = 320000
_NR = 80
_NSUB = 16
_SPT = _NR // _NSUB
_NB = _E // 128
_BPT = _NB // _NSUB
_NXTRA = _NB - _BPT * _NSUB


def _sc_segsums(ei3, zeros):
    mesh = plsc.VectorSubcoreMesh(
        core_axis_name="c", subcore_axis_name="s", num_cores=1
    )

    @functools.partial(
        pl.kernel,
        out_type=[
            jax.ShapeDtypeStruct((_NR, 128), jnp.float32),
            jax.ShapeDtypeStruct((_NR, 128), jnp.float32),
        ],
        mesh=mesh,
        compiler_params=pltpu.CompilerParams(needs_layout_passes=False),
        scratch_types=[
            pltpu.VMEM((_BPT, 2, 128), jnp.int32),
            pltpu.VMEM((1, 2, 128), jnp.int32),
            pltpu.VMEM((_NR, 128), jnp.float32),
            pltpu.VMEM((_NR, 128), jnp.float32),
            pltpu.VMEM((_SPT, 128), jnp.float32),
            pltpu.VMEM((_SPT, 128), jnp.float32),
            pltpu.VMEM((_NR,), jnp.int32),
            pltpu.VMEM_SHARED((_NR, 128), jnp.float32),
        ],
    )
    def body(ei_hbm, zeros_hbm, u_out, w_out,
             ei_t, ei_x, tbl, acc, t5a, t5b, rows, sh):
        c = lax.axis_index("c")
        s = lax.axis_index("s")
        stripe = pl.ds(s * _SPT, _SPT)
        is_leader = jnp.logical_and(s == 0, c == 0)
        has_extra = s < _NXTRA

        for b in range(_NR // 16):
            rows[pl.ds(b * 16, 16)] = lax.iota(jnp.int32, 16) + 16 * b

        pltpu.sync_copy(ei_hbm.at[pl.ds(s * _BPT, _BPT)], ei_t)

        @pl.when(has_extra)
        def _():
            pltpu.sync_copy(ei_hbm.at[pl.ds(_BPT * _NSUB + s, 1)], ei_x)

        pltpu.sync_copy(zeros_hbm, acc)

        @pl.when(s == 0)
        def _():
            pltpu.sync_copy(zeros_hbm, sh)

        plsc.subcore_barrier()

        ones16 = jnp.ones((16,), jnp.float32)

        @plsc.parallel_loop(0, _BPT, 1)
        def _(j):
            for h in range(8):
                dv = ei_t[j, 1, pl.ds(h * 16, 16)]
                plsc.addupdate_scatter(acc, [dv >> 7, dv & 127], ones16)

        @pl.when(has_extra)
        def _():
            for h in range(8):
                dv = ei_x[0, 1, pl.ds(h * 16, 16)]
                plsc.addupdate_scatter(acc, [dv >> 7, dv & 127], ones16)

        pltpu.sync_copy(acc, sh.at[rows], add=True)
        plsc.subcore_barrier()

        pltpu.sync_copy(sh.at[stripe], t5a)

        def recip_body(k, carry):
            i = k >> 3
            jc = (k & 7) * 16
            v = t5a[i, pl.ds(jc, 16)]
            rv = 1.0 / jnp.maximum(v, 1.0)
            t5a[i, pl.ds(jc, 16)] = rv
            t5b[i, pl.ds(jc, 16)] = rv
            return carry

        lax.fori_loop(0, _SPT * 8, recip_body, 0)
        pltpu.sync_copy(t5a, sh.at[stripe])
        plsc.subcore_barrier()
        pltpu.sync_copy(sh, tbl)
        pltpu.sync_copy(zeros_hbm, acc)
        plsc.subcore_barrier()

        @pl.when(s == 0)
        def _():
            pltpu.sync_copy(zeros_hbm, sh)

        plsc.subcore_barrier()

        def p_gather_body(j):
            dvs = [ei_t[j, 1, pl.ds(h * 16, 16)] for h in range(8)]
            svs = [ei_t[j, 0, pl.ds(h * 16, 16)] for h in range(8)]
            vs = [plsc.load_gather(tbl, [dv >> 7, dv & 127]) for dv in dvs]
            for h in range(8):
                sv = svs[h]
                plsc.addupdate_scatter(acc, [sv >> 7, sv & 127], vs[h])

        def p_gather_extra():
            for h in range(8):
                dv = ei_x[0, 1, pl.ds(h * 16, 16)]
                sv = ei_x[0, 0, pl.ds(h * 16, 16)]
                vs = plsc.load_gather(tbl, [dv >> 7, dv & 127])
                plsc.addupdate_scatter(acc, [sv >> 7, sv & 127], vs)

        def p_gather(_unused=None):
            plsc.parallel_loop(0, _BPT, 1)(p_gather_body)

        p_gather()
        pl.when(has_extra)(p_gather_extra)
        pltpu.sync_copy(acc, sh.at[rows], add=True)
        plsc.subcore_barrier()

        @pl.when(is_leader)
        def _():
            pltpu.sync_copy(sh, u_out)

        pltpu.sync_copy(sh.at[stripe], t5a)

        def q_body(k, carry):
            i = k >> 3
            jc = (k & 7) * 16
            t5a[i, pl.ds(jc, 16)] = t5a[i, pl.ds(jc, 16)] * t5b[i, pl.ds(jc, 16)]
            return carry

        lax.fori_loop(0, _SPT * 8, q_body, 0)
        plsc.subcore_barrier()
        pltpu.sync_copy(t5a, sh.at[stripe])
        plsc.subcore_barrier()
        pltpu.sync_copy(sh, tbl)
        pltpu.sync_copy(zeros_hbm, acc)
        plsc.subcore_barrier()

        @pl.when(s == 0)
        def _():
            pltpu.sync_copy(zeros_hbm, sh)

        plsc.subcore_barrier()

        p_gather()
        pl.when(has_extra)(p_gather_extra)
        pltpu.sync_copy(acc, sh.at[rows], add=True)
        plsc.subcore_barrier()

        @pl.when(is_leader)
        def _():
            pltpu.sync_copy(sh, w_out)

    return body(ei3, zeros)


def _tc_final(u_row, w_row, x, W1l, b1l, W1r, W2l, b2l, W2r, Wc, bc):
    def body(u_ref, w_ref, x_ref, w1l_ref, b1l_ref, w1r_ref, w2l_ref, b2l_ref,
             w2r_ref, wc_ref, bc_ref, out_ref):
        f32 = jnp.float32
        uw_rows = jnp.concatenate(
            [jnp.ones((1, _N), f32), u_ref[...], w_ref[...]], axis=0
        )
        R = jnp.dot(uw_rows, x_ref[...], preferred_element_type=f32, precision=lax.Precision.HIGHEST)
        A = jnp.dot(R, w1l_ref[...], preferred_element_type=f32, precision=lax.Precision.HIGHEST)
        B = jnp.dot(R, w1r_ref[...], preferred_element_type=f32, precision=lax.Precision.HIGHEST)
        sum_u = jnp.sum(u_ref[...])
        b1 = b1l_ref[...]
        sum_h1 = A[1:2, :] + _N * b1 + B[0:1, :]
        uth1 = A[2:3, :] + sum_u * b1 + B[1:2, :]
        T = jnp.concatenate([sum_h1, uth1], axis=0)
        C2 = jnp.dot(T, w2l_ref[...], preferred_element_type=f32, precision=lax.Precision.HIGHEST)
        D2 = jnp.dot(T, w2r_ref[...], preferred_element_type=f32, precision=lax.Precision.HIGHEST)
        sum_h2 = C2[1:2, :] + _N * b2l_ref[...] + D2[0:1, :]
        out_ref[...] = (
            jnp.dot(sum_h2 * (1.0 / _N), wc_ref[...], preferred_element_type=f32, precision=lax.Precision.HIGHEST)
            + bc_ref[...]
        )

    return pl.pallas_call(
        body,
        out_shape=jax.ShapeDtypeStruct((1, 64), jnp.float32),
    )(u_row, w_row, x, W1l, b1l, W1r, W2l, b2l, W2r, Wc, bc)


def kernel(x, edge_index, W1l, b1l, W1r, W2l, b2l, W2r, Wc, bc):
    ei3 = edge_index.reshape(2, _NB, 128).swapaxes(0, 1)
    zeros = jnp.zeros((_NR, 128), jnp.float32)
    u2d, w2d = _sc_segsums(ei3, zeros)
    u_row = u2d.reshape(1, _NR * 128)[:, :_N]
    w_row = w2d.reshape(1, _NR * 128)[:, :_N]
    return _tc_final(u_row, w_row, x, W1l, b1l[None, :], W1r, W2l,
                     b2l[None, :], W2r, Wc, bc[None, :])
